# baseline (speedup 1.0000x reference)
import jax
import jax.numpy as jnp
from jax.experimental import pallas as pl
from jax.experimental.pallas import tpu as pltpu

N = 30
K = 15
HID = 128
S = 8


def _q_head(si, p3, Wlin, blin, E3, Wuf, Wvf, Wuc, Wvc, b1, W2, b2,
            Wca, bca, Wcb, bcb):
    f32 = jnp.float32
    f = jnp.maximum(jnp.dot(si, Wlin, preferred_element_type=f32) + blin, 0.0)
    E = jnp.maximum(E3, 0.0)
    Ucf = jnp.dot(E, Wuc, preferred_element_type=f32)
    Vcf = jnp.dot(E, Wvc, preferred_element_type=f32)
    r = jax.lax.broadcasted_iota(jnp.int32, (S * N, 1), 0) % N
    is1 = jnp.logical_and(r >= 10, r < 20)
    is2 = r >= 20
    u_c = jnp.where(is2, Ucf[2:3], jnp.where(is1, Ucf[1:2], Ucf[0:1]))
    v_c = jnp.where(is2, Vcf[2:3], jnp.where(is1, Vcf[1:2], Vcf[0:1]))
    u = jnp.dot(f, Wuf, preferred_element_type=f32) + u_c + b1
    v = jnp.dot(f, Wvf, preferred_element_type=f32) + v_c

    diff = p3[:, :, None, :] - p3[:, None, :, :]
    d2 = jnp.sum(diff * diff, axis=-1)
    ii = jax.lax.broadcasted_iota(jnp.int32, (S, N, N), 1)
    jj = jax.lax.broadcasted_iota(jnp.int32, (S, N, N), 2)
    d2 = jnp.where(ii == jj, d2 + 1e10, d2)
    a = d2[:, :, :, None]
    bq = d2[:, :, None, :]
    j4 = jax.lax.broadcasted_iota(jnp.int32, (S, N, N, N), 2)
    jp4 = jax.lax.broadcasted_iota(jnp.int32, (S, N, N, N), 3)
    cmp = jnp.logical_or(bq < a, jnp.logical_and(bq == a, jp4 < j4))
    rank = jnp.sum(cmp.astype(jnp.int32), axis=-1)
    nbr = rank < K

    uu = u.reshape(S, N, 1, HID)
    vv = v.reshape(S, N, HID)[:, None, :, :]
    P = jnp.maximum(uu + vv, 0.0).astype(jnp.bfloat16)
    Z = jnp.dot(P.reshape(S * N * N, HID), W2.astype(jnp.bfloat16),
                preferred_element_type=f32) + b2
    Z = Z.reshape(S, N, N, HID)
    Zm = jnp.where(nbr[:, :, :, None], Z, -jnp.inf)
    h = jnp.maximum(jnp.max(Zm, axis=2), 0.0)
    g = jnp.maximum(
        jnp.dot(h.reshape(S * N, HID), Wca, preferred_element_type=f32) + bca,
        0.0)
    return jnp.dot(g, Wcb, preferred_element_type=f32) + bcb


def _body(si1_ref, p31_ref, si2_ref, p32_ref, Wlin1, blin1, emb1, Wif, Wic,
          Wjf, Wjc, b1, W2, b2, Wc1a, bc1a, Wc1b, bc1b, Wlin2, blin2, emb2,
          Wc2a, bc2a, Wc2b, bc2b, q1_ref, q2_ref):
    Wuf = Wif[...] - Wjf[...]
    Wuc = Wic[...] - Wjc[...]
    q1_ref[...] = _q_head(si1_ref[...], p31_ref[...], Wlin1[...], blin1[...],
                          emb1[...], Wuf, Wjf[...], Wuc, Wjc[...], b1[...],
                          W2[...], b2[...], Wc1a[...], bc1a[...], Wc1b[...],
                          bc1b[...])
    q2_ref[...] = _q_head(si2_ref[...], p32_ref[...], Wlin2[...], blin2[...],
                          emb2[...], Wuf, Wjf[...], Wuc, Wjc[...], b1[...],
                          W2[...], b2[...], Wc2a[...], bc2a[...], Wc2b[...],
                          bc2b[...])


def kernel(state, action, W_lin1, b_lin1, emb1, W_m1a, b_m1a, W_m1b, b_m1b,
           Wc1a, bc1a, Wc1b, bc1b, W_lin2, b_lin2, emb2, Wc2a, bc2a, Wc2b,
           bc2b):
    bs = state.shape[0]
    p31 = jnp.concatenate([state.reshape(bs, N, 2),
                           action.reshape(bs, N, 2)], axis=-1)
    p32 = jnp.concatenate([state, action], axis=1).reshape(bs, N, 4)
    si1 = p31.reshape(bs * N, 4)
    si2 = p32.reshape(bs * N, 4)
    Wif = W_m1a[0:128]
    Wic = W_m1a[128:192]
    Wjf = W_m1a[194:322]
    Wjc = W_m1a[322:386]

    row = lambda x: x.reshape(1, -1)
    grid = bs // S
    wspec = lambda shp: pl.BlockSpec(shp, lambda i: (0, 0))
    nspec = pl.BlockSpec((S * N, 4), lambda i: (i, 0))
    pspec = pl.BlockSpec((S, N, 4), lambda i: (i, 0, 0))
    q1, q2 = pl.pallas_call(
        _body,
        grid=(grid,),
        in_specs=[
            nspec, pspec, nspec, pspec,
            wspec((4, HID)), wspec((1, HID)), wspec((3, HID // 2)),
            wspec((128, HID)), wspec((64, HID)), wspec((128, HID)),
            wspec((64, HID)), wspec((1, HID)), wspec((HID, HID)),
            wspec((1, HID)),
            wspec((HID, HID)), wspec((1, HID)), wspec((HID, 1)), wspec((1, 1)),
            wspec((4, HID)), wspec((1, HID)), wspec((3, HID // 2)),
            wspec((HID, HID)), wspec((1, HID)), wspec((HID, 1)), wspec((1, 1)),
        ],
        out_specs=[pl.BlockSpec((S * N, 1), lambda i: (i, 0)),
                   pl.BlockSpec((S * N, 1), lambda i: (i, 0))],
        out_shape=[jax.ShapeDtypeStruct((bs * N, 1), jnp.float32),
                   jax.ShapeDtypeStruct((bs * N, 1), jnp.float32)],
        compiler_params=pltpu.CompilerParams(
            dimension_semantics=("parallel",)),
    )(si1, p31, si2, p32, W_lin1, row(b_lin1), emb1, Wif, Wic, Wjf, Wjc,
      row(b_m1a), W_m1b, row(b_m1b), Wc1a, row(bc1a), Wc1b, row(bc1b),
      W_lin2, row(b_lin2), emb2, Wc2a, row(bc2a), Wc2b, row(bc2b))
    return (q1.reshape(bs, N), q2.reshape(bs, N))


# speedup vs baseline: 2.2505x; 2.2505x over previous
import jax
import jax.numpy as jnp
from jax.experimental import pallas as pl
from jax.experimental.pallas import tpu as pltpu

N = 30
NP = 32
K = 15
HID = 128
S = 8


def _q_head(si, pN, pT, Wlin, blin, E3, Wuf, Wvf, Wuc, Wvc, b1, W2, b2,
            Wca, bca, Wcb, bcb):
    f32 = jnp.float32
    f = jnp.maximum(jnp.dot(si, Wlin, preferred_element_type=f32) + blin, 0.0)
    E = jnp.maximum(E3, 0.0)
    Ucf = jnp.dot(E, Wuc, preferred_element_type=f32)
    Vcf = jnp.dot(E, Wvc, preferred_element_type=f32)
    r = jax.lax.broadcasted_iota(jnp.int32, (S * NP, 1), 0) % NP
    is1 = jnp.logical_and(r >= 10, r < 20)
    is2 = r >= 20
    u_c = jnp.where(is2, Ucf[2:3], jnp.where(is1, Ucf[1:2], Ucf[0:1]))
    v_c = jnp.where(is2, Vcf[2:3], jnp.where(is1, Vcf[1:2], Vcf[0:1]))
    u = jnp.dot(f, Wuf, preferred_element_type=f32) + u_c + b1
    v = jnp.dot(f, Wvf, preferred_element_type=f32) + v_c

    d2 = jnp.zeros((S, NP, NP), f32)
    for d in range(4):
        diff = pN[:, :, d:d + 1] - pT[:, d:d + 1, :]
        d2 = d2 + diff * diff
    ia = jax.lax.broadcasted_iota(jnp.int32, (NP, NP), 0)
    ib = jax.lax.broadcasted_iota(jnp.int32, (NP, NP), 1)
    eye = (ia == ib)[None]
    dum = (ib >= N)[None]
    d2 = jnp.where(eye, d2 + 1e10, d2)
    d2 = jnp.where(dum, 3e10, d2)
    a = d2[:, :, :, None]
    bq = d2[:, :, None, :]
    lt = (ib < ia)[None, None]
    cond = jnp.logical_or(bq < a, jnp.logical_and(bq == a, lt))
    rank = jnp.sum(jnp.where(cond, 1.0, 0.0), axis=-1)
    nbr = rank < float(K)

    uu = u.reshape(S, NP, 1, HID)
    vv = v.reshape(S, NP, HID)[:, None, :, :]
    P = jnp.maximum(uu + vv, 0.0)
    Z = jnp.dot(P.reshape(S * NP * NP, HID), W2, preferred_element_type=f32)
    Z = Z.reshape(S, NP, NP, HID)
    Zm = jnp.where(nbr[:, :, :, None], Z, -jnp.inf)
    h = jnp.maximum(jnp.max(Zm, axis=2) + b2[None], 0.0)
    g = jnp.maximum(
        jnp.dot(h.reshape(S * NP, HID), Wca, preferred_element_type=f32)
        + bca, 0.0)
    return jnp.dot(g, Wcb, preferred_element_type=f32) + bcb


def _body(si1_ref, pN1_ref, pT1_ref, si2_ref, pN2_ref, pT2_ref, Wlin1, blin1,
          emb1, Wif, Wic, Wjf, Wjc, b1, W2, b2, Wc1a, bc1a, Wc1b, bc1b,
          Wlin2, blin2, emb2, Wc2a, bc2a, Wc2b, bc2b, q1_ref, q2_ref):
    Wuf = Wif[...] - Wjf[...]
    Wuc = Wic[...] - Wjc[...]
    q1_ref[...] = _q_head(si1_ref[...], pN1_ref[...], pT1_ref[...],
                          Wlin1[...], blin1[...], emb1[...], Wuf, Wjf[...],
                          Wuc, Wjc[...], b1[...], W2[...], b2[...],
                          Wc1a[...], bc1a[...], Wc1b[...], bc1b[...])
    q2_ref[...] = _q_head(si2_ref[...], pN2_ref[...], pT2_ref[...],
                          Wlin2[...], blin2[...], emb2[...], Wuf, Wjf[...],
                          Wuc, Wjc[...], b1[...], W2[...], b2[...],
                          Wc2a[...], bc2a[...], Wc2b[...], bc2b[...])


def kernel(state, action, W_lin1, b_lin1, emb1, W_m1a, b_m1a, W_m1b, b_m1b,
           Wc1a, bc1a, Wc1b, bc1b, W_lin2, b_lin2, emb2, Wc2a, bc2a, Wc2b,
           bc2b):
    bs = state.shape[0]
    p31 = jnp.concatenate([state.reshape(bs, N, 2),
                           action.reshape(bs, N, 2)], axis=-1)
    p32 = jnp.concatenate([state, action], axis=1).reshape(bs, N, 4)
    zpad = jnp.zeros((bs, NP - N, 4), jnp.float32)
    pN1 = jnp.concatenate([p31, zpad], axis=1)
    pN2 = jnp.concatenate([p32, zpad], axis=1)
    pT1 = pN1.transpose(0, 2, 1)
    pT2 = pN2.transpose(0, 2, 1)
    si1 = pN1.reshape(bs * NP, 4)
    si2 = pN2.reshape(bs * NP, 4)
    Wif = W_m1a[0:128]
    Wic = W_m1a[128:192]
    Wjf = W_m1a[194:322]
    Wjc = W_m1a[322:386]

    row = lambda x: x.reshape(1, -1)
    grid = bs // S
    wspec = lambda shp: pl.BlockSpec(shp, lambda i: (0, 0))
    nspec = pl.BlockSpec((S * NP, 4), lambda i: (i, 0))
    pNspec = pl.BlockSpec((S, NP, 4), lambda i: (i, 0, 0))
    pTspec = pl.BlockSpec((S, 4, NP), lambda i: (i, 0, 0))
    q1, q2 = pl.pallas_call(
        _body,
        grid=(grid,),
        in_specs=[
            nspec, pNspec, pTspec, nspec, pNspec, pTspec,
            wspec((4, HID)), wspec((1, HID)), wspec((3, HID // 2)),
            wspec((128, HID)), wspec((64, HID)), wspec((128, HID)),
            wspec((64, HID)), wspec((1, HID)), wspec((HID, HID)),
            wspec((1, HID)),
            wspec((HID, HID)), wspec((1, HID)), wspec((HID, 1)), wspec((1, 1)),
            wspec((4, HID)), wspec((1, HID)), wspec((3, HID // 2)),
            wspec((HID, HID)), wspec((1, HID)), wspec((HID, 1)), wspec((1, 1)),
        ],
        out_specs=[pl.BlockSpec((S * NP, 1), lambda i: (i, 0)),
                   pl.BlockSpec((S * NP, 1), lambda i: (i, 0))],
        out_shape=[jax.ShapeDtypeStruct((bs * NP, 1), jnp.float32),
                   jax.ShapeDtypeStruct((bs * NP, 1), jnp.float32)],
        compiler_params=pltpu.CompilerParams(
            dimension_semantics=("parallel",)),
    )(si1, pN1, pT1, si2, pN2, pT2, W_lin1, row(b_lin1), emb1, Wif, Wic, Wjf,
      Wjc, row(b_m1a), W_m1b, row(b_m1b), Wc1a, row(bc1a), Wc1b, row(bc1b),
      W_lin2, row(b_lin2), emb2, Wc2a, row(bc2a), Wc2b, row(bc2b))
    return (q1.reshape(bs, NP)[:, :N], q2.reshape(bs, NP)[:, :N])
